# baseline (speedup 1.0000x reference)
import functools

import jax
import jax.numpy as jnp
from jax import lax
from jax.experimental import pallas as pl
from jax.experimental.pallas import tpu as pltpu
from jax.experimental.pallas import tpu_sc as plsc

B = 16384
V = 100000
E = 64
H = 128
A = 8

_info = plsc.get_sparse_core_info()
NC = _info.num_cores
NS = _info.num_subcores
L = _info.num_lanes
NW = NC * NS
BPW = B // NW
CH = 128
NCH = BPW // CH

TBLK = 1024
NTBLK = pl.cdiv(V, TBLK)

_mesh = plsc.VectorSubcoreMesh(core_axis_name="c", subcore_axis_name="s")


def _transpose_body(xt_ref, o_ref):
    x = xt_ref[...]
    eye = (lax.broadcasted_iota(jnp.int32, (E, E), 0)
           == lax.broadcasted_iota(jnp.int32, (E, E), 1)).astype(jnp.float32)
    xT = lax.dot_general(x, eye, (((0,), (0,)), ((), ())),
                         precision=lax.Precision.HIGHEST,
                         preferred_element_type=jnp.float32)
    o_ref[:, 0:E] = xT
    o_ref[:, E:2 * E] = jnp.zeros((TBLK, E), jnp.float32)


def _transpose_pad(table_t):
    return pl.pallas_call(
        _transpose_body,
        grid=(NTBLK,),
        in_specs=[pl.BlockSpec((E, TBLK), lambda i: (0, i))],
        out_specs=pl.BlockSpec((TBLK, 2 * E), lambda i: (i, 0)),
        out_shape=jax.ShapeDtypeStruct((V, 2 * E), jnp.float32),
    )(table_t)


@functools.partial(
    pl.kernel,
    out_type=jax.ShapeDtypeStruct((B, 2 * E), jnp.float32),
    mesh=_mesh,
    scratch_types=[
        pltpu.VMEM((4, BPW), jnp.int32),
        pltpu.VMEM((4, CH, 2 * E), jnp.float32),
        pltpu.SemaphoreType.DMA,
    ],
)
def _gather_sum(idx_hbm, t0, t1, t2, t3, out_hbm, idx_v, buf, sem):
    wid = lax.axis_index("s") * NC + lax.axis_index("c")
    base = wid * BPW
    tables = (t0, t1, t2, t3)
    for k in range(4):
        pltpu.sync_copy(idx_hbm.at[k, pl.ds(base, BPW)], idx_v.at[k])
    for ch in range(NCH):
        cps = [
            pltpu.async_copy(
                tables[k].at[idx_v.at[k, pl.ds(ch * CH, CH)]],
                buf.at[k],
                sem,
            )
            for k in range(4)
        ]
        for cp in cps:
            cp.wait()

        def _add_row(r, carry):
            for c in range(E // L):
                s = (buf[0, r, pl.ds(c * L, L)]
                     + buf[1, r, pl.ds(c * L, L)]
                     + buf[2, r, pl.ds(c * L, L)]
                     + buf[3, r, pl.ds(c * L, L)])
                buf[0, r, pl.ds(c * L, L)] = s
            return carry

        lax.fori_loop(0, CH, _add_row, 0)
        pltpu.sync_copy(buf.at[0], out_hbm.at[pl.ds(base + ch * CH, CH)])


def _mlp_body(x_ref, w1_ref, b1_ref, w2_ref, b2_ref, wa_ref, ba_ref, o_ref):
    x = x_ref[:, 0:E]
    h = jnp.dot(x, w1_ref[...], preferred_element_type=jnp.float32) + b1_ref[...]
    h = jnp.maximum(h, 0.0)
    h = jnp.dot(h, w2_ref[...], preferred_element_type=jnp.float32) + b2_ref[...]
    h = jnp.maximum(h, 0.0)
    o_ref[...] = jnp.dot(h, wa_ref[...], preferred_element_type=jnp.float32) + ba_ref[...]


def _mlp(x, w1, b1, w2, b2, wa, ba):
    BT = 2048
    return pl.pallas_call(
        _mlp_body,
        grid=(B // BT,),
        in_specs=[
            pl.BlockSpec((BT, 2 * E), lambda i: (i, 0)),
            pl.BlockSpec((E, H), lambda i: (0, 0)),
            pl.BlockSpec((1, H), lambda i: (0, 0)),
            pl.BlockSpec((H, H), lambda i: (0, 0)),
            pl.BlockSpec((1, H), lambda i: (0, 0)),
            pl.BlockSpec((H, A), lambda i: (0, 0)),
            pl.BlockSpec((1, A), lambda i: (0, 0)),
        ],
        out_specs=pl.BlockSpec((BT, A), lambda i: (i, 0)),
        out_shape=jax.ShapeDtypeStruct((B, A), jnp.float32),
    )(x, w1, b1.reshape(1, H), w2, b2.reshape(1, H), wa, ba.reshape(1, A))


def kernel(inputs, emb_fid, emb_lba, emb_bytes, emb_bblba, w1, b1, w2, b2, wa, ba):
    idx_t = inputs.astype(jnp.int32).T
    tp = [_transpose_pad(t.T) for t in (emb_fid, emb_lba, emb_bytes, emb_bblba)]
    summed = _gather_sum(idx_t, *tp)
    return _mlp(summed, w1, b1, w2, b2, wa, ba)


# speedup vs baseline: 1.6652x; 1.6652x over previous
import functools

import jax
import jax.numpy as jnp
from jax import lax
from jax.experimental import pallas as pl
from jax.experimental.pallas import tpu as pltpu
from jax.experimental.pallas import tpu_sc as plsc

B = 16384
V = 100000
E = 64
H = 128
A = 8

_info = plsc.get_sparse_core_info()
NC = _info.num_cores
NS = _info.num_subcores
L = _info.num_lanes
NW = NC * NS
BPW = B // NW
CH = 128
NCH = BPW // CH

TBLK = 2048
NTBLK = pl.cdiv(V, TBLK)

_mesh = plsc.VectorSubcoreMesh(core_axis_name="c", subcore_axis_name="s")


def _transpose_body(xt_ref, o_ref):
    xT = lax.transpose(xt_ref[...], (1, 0))
    o_ref[:, 0:E] = xT
    o_ref[:, E:2 * E] = jnp.zeros((TBLK, E), jnp.float32)


def _transpose_pad(table_t):
    return pl.pallas_call(
        _transpose_body,
        grid=(NTBLK,),
        in_specs=[pl.BlockSpec((E, TBLK), lambda i: (0, i))],
        out_specs=pl.BlockSpec((TBLK, 2 * E), lambda i: (i, 0)),
        out_shape=jax.ShapeDtypeStruct((V, 2 * E), jnp.float32),
    )(table_t)


@functools.partial(
    pl.kernel,
    out_type=jax.ShapeDtypeStruct((B, 2 * E), jnp.float32),
    mesh=_mesh,
    scratch_types=[
        pltpu.VMEM((4, BPW), jnp.int32),
        pltpu.VMEM((4, CH, 2 * E), jnp.float32),
        pltpu.SemaphoreType.DMA,
    ],
)
def _gather_sum(idx_hbm, t0, t1, t2, t3, out_hbm, idx_v, buf, sem):
    wid = lax.axis_index("s") * NC + lax.axis_index("c")
    base = wid * BPW
    tables = (t0, t1, t2, t3)
    for k in range(4):
        pltpu.sync_copy(idx_hbm.at[k, pl.ds(base, BPW)], idx_v.at[k])
    for ch in range(NCH):
        cps = [
            pltpu.async_copy(
                tables[k].at[idx_v.at[k, pl.ds(ch * CH, CH)]],
                buf.at[k],
                sem,
            )
            for k in range(4)
        ]
        for cp in cps:
            cp.wait()

        def _add_row(r, carry):
            for c in range(E // L):
                s = (buf[0, r, pl.ds(c * L, L)]
                     + buf[1, r, pl.ds(c * L, L)]
                     + buf[2, r, pl.ds(c * L, L)]
                     + buf[3, r, pl.ds(c * L, L)])
                buf[0, r, pl.ds(c * L, L)] = s
            return carry

        lax.fori_loop(0, CH, _add_row, 0)
        pltpu.sync_copy(buf.at[0], out_hbm.at[pl.ds(base + ch * CH, CH)])


def _mlp_body(x_ref, w1_ref, b1_ref, w2_ref, b2_ref, wa_ref, ba_ref, o_ref):
    x = x_ref[:, 0:E]
    h = jnp.dot(x, w1_ref[...], preferred_element_type=jnp.float32) + b1_ref[...]
    h = jnp.maximum(h, 0.0)
    h = jnp.dot(h, w2_ref[...], preferred_element_type=jnp.float32) + b2_ref[...]
    h = jnp.maximum(h, 0.0)
    o_ref[...] = jnp.dot(h, wa_ref[...], preferred_element_type=jnp.float32) + ba_ref[...]


def _mlp(x, w1, b1, w2, b2, wa, ba):
    BT = 2048
    return pl.pallas_call(
        _mlp_body,
        grid=(B // BT,),
        in_specs=[
            pl.BlockSpec((BT, 2 * E), lambda i: (i, 0)),
            pl.BlockSpec((E, H), lambda i: (0, 0)),
            pl.BlockSpec((1, H), lambda i: (0, 0)),
            pl.BlockSpec((H, H), lambda i: (0, 0)),
            pl.BlockSpec((1, H), lambda i: (0, 0)),
            pl.BlockSpec((H, A), lambda i: (0, 0)),
            pl.BlockSpec((1, A), lambda i: (0, 0)),
        ],
        out_specs=pl.BlockSpec((BT, A), lambda i: (i, 0)),
        out_shape=jax.ShapeDtypeStruct((B, A), jnp.float32),
    )(x, w1, b1.reshape(1, H), w2, b2.reshape(1, H), wa, ba.reshape(1, A))


def kernel(inputs, emb_fid, emb_lba, emb_bytes, emb_bblba, w1, b1, w2, b2, wa, ba):
    idx_t = inputs.astype(jnp.int32).T
    tp = [_transpose_pad(t.T) for t in (emb_fid, emb_lba, emb_bytes, emb_bblba)]
    summed = _gather_sum(idx_t, *tp)
    return _mlp(summed, w1, b1, w2, b2, wa, ba)
